# Initial kernel scaffold; baseline (speedup 1.0000x reference)
import jax
import jax.numpy as jnp
from jax.experimental import pallas as pl


def kernel(x, A, S, params):
    raise NotImplementedError("write your pallas kernel here")


# speedup vs baseline: 2.8640x; 2.8640x over previous
import functools

import jax
import jax.numpy as jnp
from jax import lax
from jax.experimental import pallas as pl
from jax.experimental.pallas import tpu as pltpu

_B = 200
_V = 48
_H = 64
_NUM_LAYERS = 2
_NUM_SW = 10
_M_EDGES = (_V - 1) + _NUM_SW
_NB = 4


def _fwd_kernel(x_ref, a_ref, s_ref, emb_ref, wuv_ref, wge_ref, we12_ref,
                sW1_ref, sb1_ref, sW2T_ref, sb2_ref,
                cW1_ref, cb1_ref, cW2T_ref, cb2_ref, out_ref):
    NB, V, H = _NB, _V, _H
    f32 = jnp.float32

    A3 = a_ref[...]
    S3 = s_ref[...]
    x_cur = x_ref[...]

    mask = ((A3 + S3) > 0).astype(f32)
    deg = jnp.sum(mask, axis=2, keepdims=True) + 1e-6
    inv_deg = 1.0 / deg

    e0 = emb_ref[0:1, :]
    e1 = emb_ref[1:2, :]
    s = e0[None] + S3[..., None] * (e1 - e0)[None]

    for l in range(_NUM_LAYERS):
        uv = jnp.dot(x_cur.reshape(NB * V, H), wuv_ref[l],
                     preferred_element_type=f32)
        Ux = uv[:, :H].reshape(NB, V, H)
        Vx = uv[:, H:].reshape(NB, V, H)
        ge = jnp.dot(s.reshape(NB * V * V, H), wge_ref[l],
                     preferred_element_type=f32)
        gates = jax.nn.sigmoid(ge[:, :H]).reshape(NB, V, V, H)
        sE0 = ge[:, H:].reshape(NB, V, V, H)
        msg = gates * Vx[:, None, :, :] * mask[..., None]
        agg = jnp.sum(msg, axis=2) * inv_deg
        x_cur = jnp.maximum(Ux + agg, 0.0)
        e12 = jnp.dot(x_cur.reshape(NB * V, H), we12_ref[l],
                      preferred_element_type=f32).reshape(NB, V, 2 * H)
        xE1 = e12[:, :, :H]
        xE2 = e12[:, :, H:]
        s = jnp.maximum(sE0 + xE1[:, :, None, :] + xE2[:, None, :, :], 0.0)

    xg = jnp.sum(x_cur, axis=1)

    ii = lax.broadcasted_iota(jnp.int32, (V, V), 0)
    jj = lax.broadcasted_iota(jnp.int32, (V, V), 1)
    triu = (jj > ii).astype(f32)
    csum_right = (ii <= jj).astype(f32)
    csum_below = (jj < ii).astype(f32)

    kS3 = lax.broadcasted_iota(jnp.int32, (16, 1, 1), 0).astype(f32) + 1.0
    kA3 = lax.broadcasted_iota(jnp.int32, (V, 1, 1), 0).astype(f32) + 1.0
    flatpos = (ii * V + jj).astype(f32)[None]
    piota = lax.broadcasted_iota(jnp.int32, (16, V * V), 1).astype(f32)
    s_flat = s.reshape(NB * V * V, H)

    for nb in range(NB):
        s_nb = s_flat[nb * V * V:(nb + 1) * V * V]
        x_nb = x_cur[nb]
        xg_nb = xg[nb:nb + 1, :]

        tS = S3[nb] * triu
        crow = jnp.dot(tS, csum_right, preferred_element_type=f32)
        rtot = crow[:, V - 1:V]
        roff = jnp.dot(csum_below, rtot, preferred_element_type=f32)
        cS = crow + roff
        oh3 = jnp.where(jnp.abs(cS[None] - kS3) < 0.5, tS[None], 0.0)
        idx = jnp.sum(jnp.sum(oh3 * flatpos, axis=2), axis=1,
                      keepdims=True)
        ohS = (jnp.abs(idx - piota) < 0.5).astype(f32)
        sw = jnp.dot(ohS, s_nb, preferred_element_type=f32)
        Ri = jnp.sum(oh3, axis=2)
        Rj = jnp.sum(oh3, axis=1)
        x1 = jnp.dot(Ri, x_nb, preferred_element_type=f32)
        x2 = jnp.dot(Rj, x_nb, preferred_element_type=f32)
        smlp_in = jnp.concatenate(
            [sw, x1, x2, jnp.broadcast_to(xg_nb, (16, H))], axis=1)
        hs = jnp.maximum(
            jnp.dot(smlp_in, sW1_ref[...], preferred_element_type=f32)
            + sb1_ref[...], 0.0)
        sT = lax.dot_general(sW2T_ref[...], hs, (((1,), (1,)), ((), ())),
                             preferred_element_type=f32)
        sT = sT + sb2_ref[...]

        tA = A3[nb] * triu
        crowA = jnp.dot(tA, csum_right, preferred_element_type=f32)
        rtotA = crowA[:, V - 1:V]
        roffA = jnp.dot(csum_below, rtotA, preferred_element_type=f32)
        cA = crowA + roffA
        ohA3 = jnp.where(jnp.abs(cA[None] - kA3) < 0.5, tA[None], 0.0)
        Bi = jnp.sum(ohA3, axis=2)
        Bj = jnp.sum(ohA3, axis=1)
        xb = jnp.dot(Bi, x_nb, preferred_element_type=f32)
        xe = jnp.dot(Bj, x_nb, preferred_element_type=f32)
        cmlp_in = jnp.concatenate(
            [xb, xe, jnp.broadcast_to(xg_nb, (V, H))], axis=1)
        hc = jnp.maximum(
            jnp.dot(cmlp_in, cW1_ref[...], preferred_element_type=f32)
            + cb1_ref[...], 0.0)
        cT = lax.dot_general(cW2T_ref[...], hc, (((1,), (1,)), ((), ())),
                             preferred_element_type=f32)
        cT = cT + cb2_ref[...]

        nsw = _NUM_SW
        nbr = _V - 1
        zeros47 = jnp.zeros((1, nbr), f32)
        p_flow = jnp.concatenate([cT[0:1, :nbr], sT[1:2, :nsw]], axis=1)
        topo = jnp.concatenate([zeros47, jax.nn.sigmoid(sT[0:1, :nsw])], axis=1)
        v_par = jnp.concatenate([cT[1:2, :nbr], sT[2:3, :nsw]], axis=1)
        v_chd = jnp.concatenate([cT[2:3, :nbr], sT[3:4, :nsw]], axis=1)
        row = jnp.concatenate([p_flow, topo, v_par, v_chd], axis=1)
        out_ref[0, nb:nb + 1, :] = row


@jax.jit
def kernel(x, A, S, params):
    f32 = jnp.float32
    H = _H
    lp = params['layers']
    wuv = jnp.stack([jnp.concatenate([l['U'], l['Vm']], axis=1) for l in lp])
    wge = jnp.stack([jnp.concatenate([l['G'], l['E0']], axis=1) for l in lp])
    we12 = jnp.stack([jnp.concatenate([l['E1'], l['E2']], axis=1) for l in lp])
    emb = params['embed']
    sW1 = params['smlp_W1']
    sb1 = params['smlp_b1'].reshape(1, 4 * H)
    sW2T = jnp.zeros((8, 4 * H), f32).at[:4].set(params['smlp_W2'].T)
    sb2 = jnp.zeros((8, 1), f32).at[:4, 0].set(params['smlp_b2'])
    cW1 = params['cmlp_W1']
    cb1 = params['cmlp_b1'].reshape(1, 3 * H)
    cW2T = jnp.zeros((8, 3 * H), f32).at[:3].set(params['cmlp_W2'].T)
    cb2 = jnp.zeros((8, 1), f32).at[:3, 0].set(params['cmlp_b2'])

    grid = (_B // _NB,)
    full = lambda shape: pl.BlockSpec(shape, lambda i: (0,) * len(shape))
    out = pl.pallas_call(
        _fwd_kernel,
        grid=grid,
        in_specs=[
            pl.BlockSpec((_NB, _V, _H), lambda i: (i, 0, 0)),
            pl.BlockSpec((_NB, _V, _V), lambda i: (i, 0, 0)),
            pl.BlockSpec((_NB, _V, _V), lambda i: (i, 0, 0)),
            full((2, H)),
            full((_NUM_LAYERS, H, 2 * H)),
            full((_NUM_LAYERS, H, 2 * H)),
            full((_NUM_LAYERS, H, 2 * H)),
            full((4 * H, 4 * H)),
            full((1, 4 * H)),
            full((8, 4 * H)),
            full((8, 1)),
            full((3 * H, 3 * H)),
            full((1, 3 * H)),
            full((8, 3 * H)),
            full((8, 1)),
        ],
        out_specs=pl.BlockSpec((1, _NB, 4 * _M_EDGES), lambda i: (i, 0, 0)),
        out_shape=jax.ShapeDtypeStruct((_B // _NB, _NB, 4 * _M_EDGES), f32),
        compiler_params=pltpu.CompilerParams(
            dimension_semantics=("parallel",)),
    )(x, A, S, emb, wuv, wge, we12, sW1, sb1, sW2T, sb2, cW1, cb1, cW2T, cb2)
    return out.reshape(_B, 4 * _M_EDGES)
